# Initial kernel scaffold; baseline (speedup 1.0000x reference)
import jax
import jax.numpy as jnp
from jax.experimental import pallas as pl


def kernel(categorical_features, continous_features, emb_table, W0, b0, W1, b1, W2, b2, W3, b3, W4, b4):
    raise NotImplementedError("write your pallas kernel here")


# speedup vs baseline: 14.3062x; 14.3062x over previous
import functools

import jax
import jax.numpy as jnp
from jax import lax
from jax.experimental import pallas as pl
from jax.experimental.pallas import tpu as pltpu
from jax.experimental.pallas import tpu_sc as plsc

B = 16384
NF = 26
EMD = 32
FEAT = NF * EMD
CONT = 13
WPACK = 848

NC = 2
NS = 16
NW = NC * NS
ROWS_W = B // NW
CHUNK = 32
NCHUNK = ROWS_W // CHUNK
IDX_PER_CHUNK = CHUNK * NF
GSIZE = 64
NSTREAM = IDX_PER_CHUNK // GSIZE
TROW = 128
TBL_ROWS = 1000000 * EMD // TROW


def _fold_body(w0, w1, w2, w3, w4, b0, b1, b2, b3, b4, out_ref):
    hp = jax.lax.Precision.HIGHEST
    v4 = w4[...]
    v3 = jnp.dot(w3[...], v4, precision=hp)
    v2 = jnp.dot(w2[...], v3, precision=hp)
    v1 = jnp.dot(w1[...], v2, precision=hp)
    wv = jnp.dot(w0[...], v1, precision=hp)
    c = (jnp.dot(b0[...], v1, precision=hp)
         + jnp.dot(b1[...], v2, precision=hp)
         + jnp.dot(b2[...], v3, precision=hp)
         + jnp.dot(b3[...], v4, precision=hp)
         + b4[...])
    m = lax.broadcasted_iota(jnp.int32, (WPACK, 1), 0) == (FEAT + CONT)
    out_ref[...] = wv + jnp.where(m, c, jnp.zeros_like(c))


_fold = pl.pallas_call(
    _fold_body,
    out_shape=jax.ShapeDtypeStruct((WPACK, 1), jnp.float32),
)


def _sc_body(cat_hbm, cont_hbm, tab_hbm, w_hbm, out_hbm,
             idx_v, idxc_v, buf_v, cont_v, w_v, out_v, sem):
    wid = lax.axis_index("s") * NC + lax.axis_index("c")
    pltpu.sync_copy(w_hbm, w_v)
    lanes = lax.iota(jnp.int32, 16)

    def chunk_body(t, carry):
        pltpu.sync_copy(cat_hbm.at[wid, t], idx_v)
        for q in range(NSTREAM):
            for kk in range(GSIZE // 16):
                v = idx_v[q, pl.ds(kk * 16, 16)]
                idxc_v[q, pl.ds(kk * 16, 16)] = v >> 2
        cps = [pltpu.async_copy(tab_hbm.at[idxc_v.at[q]],
                                buf_v.at[pl.ds(q * GSIZE, GSIZE)], sem)
               for q in range(NSTREAM)]
        pltpu.sync_copy(cont_hbm.at[wid, t], cont_v)
        for c in cps:
            c.wait()
        wtail = w_v[pl.ds(FEAT, 16)]

        def group_body(g, carry2):
            i0 = g * 16
            p0 = (i0 + lanes) * NF
            acc0 = jnp.full((16,), wtail[CONT], jnp.float32)

            def f_body(f, acc):
                p = p0 + f
                raw = plsc.load_gather(idx_v, [p >> 6, p & (GSIZE - 1)])
                off = (raw & 3) * EMD
                wlo = w_v[pl.ds(f * EMD, 16)]
                whi = w_v[pl.ds(f * EMD + 16, 16)]
                for j in range(16):
                    vals = plsc.load_gather(buf_v, [p, off + j])
                    acc = acc + vals * wlo[j]
                for j in range(16):
                    vals = plsc.load_gather(buf_v, [p, off + 16 + j])
                    acc = acc + vals * whi[j]
                return acc

            acc = lax.fori_loop(0, NF, f_body, acc0)
            cb = (i0 + lanes) * CONT
            for j in range(CONT):
                cv = plsc.load_gather(cont_v, [cb + j])
                acc = acc + cv * wtail[j]
            y = 1.0 / (1.0 + jnp.exp(-acc))
            out_v[pl.ds(t * CHUNK + i0, 16)] = y
            return carry2

        lax.fori_loop(0, CHUNK // 16, group_body, 0)
        return carry

    lax.fori_loop(0, NCHUNK, chunk_body, 0)
    pltpu.sync_copy(out_v, out_hbm.at[pl.ds(wid * ROWS_W, ROWS_W)])


_sc_classify = functools.partial(
    pl.kernel,
    mesh=plsc.VectorSubcoreMesh(core_axis_name="c", subcore_axis_name="s"),
    compiler_params=pltpu.CompilerParams(needs_layout_passes=False),
    out_type=jax.ShapeDtypeStruct((B,), jnp.float32),
    scratch_types=[
        pltpu.VMEM((NSTREAM, GSIZE), jnp.int32),
        pltpu.VMEM((NSTREAM, GSIZE), jnp.int32),
        pltpu.VMEM((IDX_PER_CHUNK, TROW), jnp.float32),
        pltpu.VMEM((CHUNK * CONT,), jnp.float32),
        pltpu.VMEM((WPACK,), jnp.float32),
        pltpu.VMEM((ROWS_W,), jnp.float32),
        pltpu.SemaphoreType.DMA,
    ],
)(_sc_body)


def kernel(categorical_features, continous_features, emb_table,
           W0, b0, W1, b1, W2, b2, W3, b3, W4, b4):
    cat32 = categorical_features.astype(jnp.int32).reshape(
        NW, NCHUNK, NSTREAM, GSIZE)
    cont_r = continous_features.reshape(NW, NCHUNK, CHUNK * CONT)
    tab = emb_table.reshape(TBL_ROWS, TROW)
    w0p = jnp.pad(W0, ((0, WPACK - W0.shape[0]), (0, 0)))
    wpack = _fold(w0p, W1, W2, W3, W4,
                  b0.reshape(1, -1), b1.reshape(1, -1), b2.reshape(1, -1),
                  b3.reshape(1, -1), b4.reshape(1, -1))
    out = _sc_classify(cat32, cont_r, tab, wpack.reshape(WPACK))
    return out.reshape(B, 1)
